# baseline (speedup 1.0000x reference)
import jax
import jax.numpy as jnp
from jax.experimental import pallas as pl
from jax.experimental.pallas import tpu as pltpu

_NC = 80
_B = 8
_G = 20
_A = 8576
_BIG_I = 1 << 30
_NK = 10


def _loss_body(f0, f1, f2, lab, cid, out, fs, dm):
    b = pl.program_id(0)
    fs[:, 0:6400] = f0[0]
    fs[:, 6400:8000] = f1[0]
    fs[:, 8064:8464] = f2[0]
    fs[:, 8000:8064] = jnp.zeros((85, 64), jnp.float32)
    fs[:, 8464:8576] = jnp.zeros((85, 112), jnp.float32)

    a_i = jax.lax.broadcasted_iota(jnp.int32, (1, _A), 1)
    lvl0 = a_i < 6400
    in01 = a_i < 8064
    valid = lvl0 | ((a_i >= 6400) & (a_i < 8000)) | ((a_i >= 8064) & (a_i < 8464))
    stride = jnp.where(lvl0, 8.0, jnp.where(in01, 16.0, 32.0))
    local = jnp.where(lvl0, a_i, jnp.where(in01, a_i - 6400, a_i - 8064)).astype(jnp.float32)
    wdt = jnp.where(lvl0, 80.0, jnp.where(in01, 40.0, 20.0))
    gy = jnp.floor((local + 0.5) / wdt)
    gx = local - gy * wdt
    xc = (gx + 0.5) * stride
    yc = (gy + 0.5) * stride

    xr = fs[0:1, :]
    yr = fs[1:2, :]
    wr = fs[2:3, :]
    hr = fs[3:4, :]
    ob = fs[4:5, :]
    cls = fs[5:85, :]

    bx = (xr + gx) * stride
    by = (yr + gy) * stride
    bw = jnp.exp(wr) * stride
    bh = jnp.exp(hr) * stride

    e = jnp.exp(-jnp.abs(cls))
    l1pe = jnp.log1p(e)
    sbce = jnp.sum(jnp.maximum(cls, 0.0) + l1pe, axis=0, keepdims=True)
    lsc = jnp.minimum(cls, 0.0) - l1pe
    lso = jnp.minimum(ob, 0.0) - jnp.log1p(jnp.exp(-jnp.abs(ob)))
    logs = 0.5 * (lsc + lso)
    s = jnp.exp(logs)
    log1ms = jnp.log(1.0 - s + 1e-8)
    l0 = jnp.sum(log1ms, axis=0, keepdims=True)
    dm[...] = logs - log1ms

    labv = lab[0]
    gtx = labv[:, 0:1]
    gty = labv[:, 1:2]
    gtw = labv[:, 2:3]
    gth = labv[:, 3:4]

    rows_d = []
    rows_c = []
    for g in range(_G):
        c = cid[0, 0, g]
        rows_d.append(dm[pl.ds(c, 1), :])
        rows_c.append(fs[pl.ds(5 + c, 1), :])
    dsel = jnp.concatenate(rows_d, axis=0)
    cpsel = jnp.concatenate(rows_c, axis=0)

    in_boxes = ((xc > gtx - 0.5 * gtw) & (xc < gtx + 0.5 * gtw)
                & (yc > gty - 0.5 * gth) & (yc < gty + 0.5 * gth))
    in_centers = ((xc > gtx - 2.5 * stride) & (xc < gtx + 2.5 * stride)
                  & (yc > gty - 2.5 * stride) & (yc < gty + 2.5 * stride))
    fg = (jnp.max((in_boxes | in_centers).astype(jnp.float32), axis=0,
                  keepdims=True) > 0.0) & valid
    in_both = in_boxes & in_centers

    tlx = jnp.maximum(gtx - 0.5 * gtw, bx - 0.5 * bw)
    tly = jnp.maximum(gty - 0.5 * gth, by - 0.5 * bh)
    brx = jnp.minimum(gtx + 0.5 * gtw, bx + 0.5 * bw)
    bry = jnp.minimum(gty + 0.5 * gth, by + 0.5 * bh)
    en = ((tlx < brx) & (tly < bry)).astype(jnp.float32)
    area_i = (brx - tlx) * (bry - tly) * en
    ious = area_i / (gtw * gth + bw * bh - area_i + 1e-16)

    iou_cost = -jnp.log(ious + 1e-8)
    cls_cost = -(l0 + dsel)
    cost = (cls_cost + 3.0 * iou_cost
            + 100000.0 * (1.0 - in_both.astype(jnp.float32))
            + 100000.0 * (1.0 - fg.astype(jnp.float32)))
    cost = jnp.where(valid, cost, 1e30)

    cur = jnp.concatenate([jnp.where(fg, ious, 0.0), -cost], axis=0)
    ksum = jnp.zeros((_G, 1), jnp.float32)
    idxs = []
    for _ in range(_NK):
        m = jnp.max(cur, axis=1, keepdims=True)
        ksum = ksum + m[0:_G]
        idx = jnp.min(jnp.where(cur == m, a_i, _BIG_I), axis=1, keepdims=True)
        idxs.append(idx[_G:])
        cur = jnp.where(a_i == idx, -1e35, cur)
    dyn_k = jnp.clip(ksum.astype(jnp.int32), 1, _NK)

    matchf = jnp.zeros((_G, _A), jnp.float32)
    for j in range(_NK):
        matchf = jnp.maximum(
            matchf, jnp.where((a_i == idxs[j]) & (dyn_k > j), 1.0, 0.0))

    amg = jnp.sum(matchf, axis=0, keepdims=True)
    minc = jnp.min(cost, axis=0, keepdims=True)
    g_i = jax.lax.broadcasted_iota(jnp.int32, (_G, _A), 0)
    garg = jnp.min(jnp.where(cost == minc, g_i, 99), axis=0, keepdims=True)
    matchf = jnp.where(amg > 1.0, (g_i == garg).astype(jnp.float32), matchf)

    fgf = jnp.max(matchf, axis=0, keepdims=True)
    pious = jnp.sum(matchf * ious, axis=0, keepdims=True)
    tx = jnp.sum(matchf * gtx, axis=0, keepdims=True)
    ty = jnp.sum(matchf * gty, axis=0, keepdims=True)
    tw = jnp.sum(matchf * gtw, axis=0, keepdims=True)
    th = jnp.sum(matchf * gth, axis=0, keepdims=True)
    csel = jnp.sum(matchf * cpsel, axis=0, keepdims=True)

    tlx2 = jnp.maximum(bx - 0.5 * bw, tx - 0.5 * tw)
    tly2 = jnp.maximum(by - 0.5 * bh, ty - 0.5 * th)
    brx2 = jnp.minimum(bx + 0.5 * bw, tx + 0.5 * tw)
    bry2 = jnp.minimum(by + 0.5 * bh, ty + 0.5 * th)
    en2 = ((tlx2 < brx2) & (tly2 < bry2)).astype(jnp.float32)
    ai2 = (brx2 - tlx2) * (bry2 - tly2) * en2
    iou2 = ai2 / (bw * bh + tw * th - ai2 + 1e-16)
    t_iou = jnp.sum((1.0 - iou2 * iou2) * fgf)

    bce_obj = jnp.maximum(ob, 0.0) - ob * fgf + jnp.log1p(jnp.exp(-jnp.abs(ob)))
    t_obj = jnp.sum(jnp.where(valid, bce_obj, 0.0))
    t_cls = jnp.sum(fgf * (sbce - csel * pious))
    t_fg = jnp.sum(fgf)

    li = jax.lax.broadcasted_iota(jnp.int32, (1, 8), 1)
    vec = (jnp.where(li == 0, t_iou, 0.0) + jnp.where(li == 1, t_obj, 0.0)
           + jnp.where(li == 2, t_cls, 0.0) + jnp.where(li == 3, t_fg, 0.0))

    @pl.when(b == 0)
    def _():
        out[...] = vec

    @pl.when(b > 0)
    def _():
        out[...] = out[...] + vec

    @pl.when(b == _B - 1)
    def _():
        acc = out[...]
        num_fg = jnp.maximum(jnp.sum(jnp.where(li == 3, acc, 0.0)), 1.0)
        total = (5.0 * jnp.sum(jnp.where(li == 0, acc, 0.0))
                 + jnp.sum(jnp.where(li == 1, acc, 0.0))
                 + jnp.sum(jnp.where(li == 2, acc, 0.0)))
        out[...] = jnp.where(li == 4, total / num_fg, acc)


def kernel(feat0, feat1, feat2, labels):
    f0 = feat0.reshape(_B, 85, 6400)
    f1 = feat1.reshape(_B, 85, 1600)
    f2 = feat2.reshape(_B, 85, 400)
    cls_ids = labels[:, :, 4].astype(jnp.int32).reshape(_B, 1, _G)
    out = pl.pallas_call(
        _loss_body,
        grid=(_B,),
        in_specs=[
            pl.BlockSpec((1, 85, 6400), lambda b: (b, 0, 0)),
            pl.BlockSpec((1, 85, 1600), lambda b: (b, 0, 0)),
            pl.BlockSpec((1, 85, 400), lambda b: (b, 0, 0)),
            pl.BlockSpec((1, _G, 5), lambda b: (b, 0, 0)),
            pl.BlockSpec((1, 1, _G), lambda b: (b, 0, 0), memory_space=pltpu.SMEM),
        ],
        out_specs=pl.BlockSpec((1, 8), lambda b: (0, 0)),
        out_shape=jax.ShapeDtypeStruct((1, 8), jnp.float32),
        scratch_shapes=[pltpu.VMEM((85, _A), jnp.float32),
                        pltpu.VMEM((_NC, _A), jnp.float32)],
    )(f0, f1, f2, labels, cls_ids)
    return out[0, 4]


# speedup vs baseline: 1.2960x; 1.2960x over previous
import jax
import jax.numpy as jnp
from jax.experimental import pallas as pl
from jax.experimental.pallas import tpu as pltpu

_NC = 80
_B = 8
_G = 20
_A = 8576
_BIG_I = 1 << 30
_NK = 10


def _loss_body(f0, f1, f2, lab, out, fs):
    b = pl.program_id(0)
    fs[:, 0:6400] = f0[0]
    fs[:, 6400:8000] = f1[0]
    fs[:, 8064:8464] = f2[0]
    fs[:, 8000:8064] = jnp.zeros((85, 64), jnp.float32)
    fs[:, 8464:8576] = jnp.zeros((85, 112), jnp.float32)

    a_i = jax.lax.broadcasted_iota(jnp.int32, (1, _A), 1)
    lvl0 = a_i < 6400
    in01 = a_i < 8064
    valid = lvl0 | ((a_i >= 6400) & (a_i < 8000)) | ((a_i >= 8064) & (a_i < 8464))
    stride = jnp.where(lvl0, 8.0, jnp.where(in01, 16.0, 32.0))
    local = jnp.where(lvl0, a_i, jnp.where(in01, a_i - 6400, a_i - 8064)).astype(jnp.float32)
    wdt = jnp.where(lvl0, 80.0, jnp.where(in01, 40.0, 20.0))
    gy = jnp.floor((local + 0.5) / wdt)
    gx = local - gy * wdt
    xc = (gx + 0.5) * stride
    yc = (gy + 0.5) * stride

    xr = fs[0:1, :]
    yr = fs[1:2, :]
    wr = fs[2:3, :]
    hr = fs[3:4, :]
    ob = fs[4:5, :]
    cls = fs[5:85, :]

    bx = (xr + gx) * stride
    by = (yr + gy) * stride
    bw = jnp.exp(wr) * stride
    bh = jnp.exp(hr) * stride

    e = jnp.exp(-jnp.abs(cls))
    l1pe = jnp.log1p(e)
    sbce = jnp.sum(jnp.maximum(cls, 0.0) + l1pe, axis=0, keepdims=True)
    lsc = jnp.minimum(cls, 0.0) - l1pe
    lso = jnp.minimum(ob, 0.0) - jnp.log1p(jnp.exp(-jnp.abs(ob)))
    logs = 0.5 * (lsc + lso)
    s = jnp.exp(logs)
    log1ms = jnp.log(1.0 - s + 1e-8)
    l0 = jnp.sum(log1ms, axis=0, keepdims=True)
    dmat = logs - log1ms

    labv = lab[0]
    gtx = labv[:, 0:1]
    gty = labv[:, 1:2]
    gtw = labv[:, 2:3]
    gth = labv[:, 3:4]

    gcls = labv[:, 4:5]
    c_i = jax.lax.broadcasted_iota(jnp.int32, (_G, _NC), 1)
    onehot = (gcls.astype(jnp.int32) == c_i).astype(jnp.float32)
    dsel = jax.lax.dot_general(onehot, dmat, (((1,), (0,)), ((), ())),
                               precision=jax.lax.Precision.HIGHEST,
                               preferred_element_type=jnp.float32)
    cpsel = jax.lax.dot_general(onehot, cls, (((1,), (0,)), ((), ())),
                                precision=jax.lax.Precision.HIGHEST,
                                preferred_element_type=jnp.float32)

    in_boxes = ((xc > gtx - 0.5 * gtw) & (xc < gtx + 0.5 * gtw)
                & (yc > gty - 0.5 * gth) & (yc < gty + 0.5 * gth))
    in_centers = ((xc > gtx - 2.5 * stride) & (xc < gtx + 2.5 * stride)
                  & (yc > gty - 2.5 * stride) & (yc < gty + 2.5 * stride))
    fg = (jnp.max((in_boxes | in_centers).astype(jnp.float32), axis=0,
                  keepdims=True) > 0.0) & valid
    in_both = in_boxes & in_centers

    tlx = jnp.maximum(gtx - 0.5 * gtw, bx - 0.5 * bw)
    tly = jnp.maximum(gty - 0.5 * gth, by - 0.5 * bh)
    brx = jnp.minimum(gtx + 0.5 * gtw, bx + 0.5 * bw)
    bry = jnp.minimum(gty + 0.5 * gth, by + 0.5 * bh)
    en = ((tlx < brx) & (tly < bry)).astype(jnp.float32)
    area_i = (brx - tlx) * (bry - tly) * en
    ious = area_i / (gtw * gth + bw * bh - area_i + 1e-16)

    iou_cost = -jnp.log(ious + 1e-8)
    cls_cost = -(l0 + dsel)
    penalty = jnp.where(valid & in_both, 0.0, 100000.0) + jnp.where(
        valid & fg, 0.0, jnp.where(valid, 100000.0, 1e30))
    cost = cls_cost + 3.0 * iou_cost + penalty

    a_f = jax.lax.broadcasted_iota(jnp.int32, (1, _A), 1).astype(jnp.float32)
    cur = jnp.concatenate([jnp.where(fg, ious, 0.0), -cost], axis=0)
    ksum = jnp.zeros((_G, 1), jnp.float32)
    mcs, idcs = [], []
    for _ in range(_NK):
        m = jnp.max(cur, axis=1, keepdims=True)
        ksum = ksum + m[0:_G]
        idx = jnp.min(jnp.where(cur == m, a_f, 3e8), axis=1, keepdims=True)
        mcs.append(m[_G:])
        idcs.append(idx[_G:])
        cur = jnp.where(a_f == idx, -1e35, cur)
    dyn_k = jnp.clip(ksum.astype(jnp.int32), 1, _NK)

    thr_v = jnp.zeros((_G, 1), jnp.float32)
    thr_i = jnp.zeros((_G, 1), jnp.float32)
    for j in range(_NK):
        sel = (dyn_k == j + 1).astype(jnp.float32)
        thr_v = thr_v + sel * (-mcs[j])
        thr_i = thr_i + sel * idcs[j]
    matchf = jnp.where(
        (cost < thr_v) | ((cost == thr_v) & (a_f <= thr_i)), 1.0, 0.0)

    amg = jnp.sum(matchf, axis=0, keepdims=True)
    minc = jnp.min(cost, axis=0, keepdims=True)
    g_f = jax.lax.broadcasted_iota(jnp.int32, (_G, _A), 0).astype(jnp.float32)
    garg = jnp.min(jnp.where(cost == minc, g_f, 99.0), axis=0, keepdims=True)
    matchf = jnp.where(amg > 1.0, jnp.where(g_f == garg, 1.0, 0.0), matchf)

    fgf = jnp.max(matchf, axis=0, keepdims=True)
    pious = jnp.sum(matchf * ious, axis=0, keepdims=True)
    tx = jnp.sum(matchf * gtx, axis=0, keepdims=True)
    ty = jnp.sum(matchf * gty, axis=0, keepdims=True)
    tw = jnp.sum(matchf * gtw, axis=0, keepdims=True)
    th = jnp.sum(matchf * gth, axis=0, keepdims=True)
    csel = jnp.sum(matchf * cpsel, axis=0, keepdims=True)

    tlx2 = jnp.maximum(bx - 0.5 * bw, tx - 0.5 * tw)
    tly2 = jnp.maximum(by - 0.5 * bh, ty - 0.5 * th)
    brx2 = jnp.minimum(bx + 0.5 * bw, tx + 0.5 * tw)
    bry2 = jnp.minimum(by + 0.5 * bh, ty + 0.5 * th)
    en2 = ((tlx2 < brx2) & (tly2 < bry2)).astype(jnp.float32)
    ai2 = (brx2 - tlx2) * (bry2 - tly2) * en2
    iou2 = ai2 / (bw * bh + tw * th - ai2 + 1e-16)
    t_iou = jnp.sum((1.0 - iou2 * iou2) * fgf)

    bce_obj = jnp.maximum(ob, 0.0) - ob * fgf + jnp.log1p(jnp.exp(-jnp.abs(ob)))
    t_obj = jnp.sum(jnp.where(valid, bce_obj, 0.0))
    t_cls = jnp.sum(fgf * (sbce - csel * pious))
    t_fg = jnp.sum(fgf)

    li = jax.lax.broadcasted_iota(jnp.int32, (1, 8), 1)
    vec = (jnp.where(li == 0, t_iou, 0.0) + jnp.where(li == 1, t_obj, 0.0)
           + jnp.where(li == 2, t_cls, 0.0) + jnp.where(li == 3, t_fg, 0.0))

    @pl.when(b == 0)
    def _():
        out[...] = vec

    @pl.when(b > 0)
    def _():
        out[...] = out[...] + vec

    @pl.when(b == _B - 1)
    def _():
        acc = out[...]
        num_fg = jnp.maximum(jnp.sum(jnp.where(li == 3, acc, 0.0)), 1.0)
        total = (5.0 * jnp.sum(jnp.where(li == 0, acc, 0.0))
                 + jnp.sum(jnp.where(li == 1, acc, 0.0))
                 + jnp.sum(jnp.where(li == 2, acc, 0.0)))
        out[...] = jnp.where(li == 4, total / num_fg, acc)


def kernel(feat0, feat1, feat2, labels):
    f0 = feat0.reshape(_B, 85, 6400)
    f1 = feat1.reshape(_B, 85, 1600)
    f2 = feat2.reshape(_B, 85, 400)
    out = pl.pallas_call(
        _loss_body,
        grid=(_B,),
        in_specs=[
            pl.BlockSpec((1, 85, 6400), lambda b: (b, 0, 0)),
            pl.BlockSpec((1, 85, 1600), lambda b: (b, 0, 0)),
            pl.BlockSpec((1, 85, 400), lambda b: (b, 0, 0)),
            pl.BlockSpec((1, _G, 5), lambda b: (b, 0, 0)),
        ],
        out_specs=pl.BlockSpec((1, 8), lambda b: (0, 0)),
        out_shape=jax.ShapeDtypeStruct((1, 8), jnp.float32),
        scratch_shapes=[pltpu.VMEM((85, _A), jnp.float32)],
    )(f0, f1, f2, labels)
    return out[0, 4]


# speedup vs baseline: 1.3626x; 1.0514x over previous
import jax
import jax.numpy as jnp
from jax.experimental import pallas as pl
from jax.experimental.pallas import tpu as pltpu

_NC = 80
_B = 8
_G = 20
_A = 8576
_BIG_I = 1 << 30
_NK = 10


def _loss_body(f0, f1, f2, lab, out, fs):
    b = pl.program_id(0)
    fs[:, 0:6400] = f0[0]
    fs[:, 6400:8000] = f1[0]
    fs[:, 8064:8464] = f2[0]
    fs[:, 8000:8064] = jnp.zeros((85, 64), jnp.float32)
    fs[:, 8464:8576] = jnp.zeros((85, 112), jnp.float32)

    a_i = jax.lax.broadcasted_iota(jnp.int32, (1, _A), 1)
    lvl0 = a_i < 6400
    in01 = a_i < 8064
    valid = lvl0 | ((a_i >= 6400) & (a_i < 8000)) | ((a_i >= 8064) & (a_i < 8464))
    stride = jnp.where(lvl0, 8.0, jnp.where(in01, 16.0, 32.0))
    local = jnp.where(lvl0, a_i, jnp.where(in01, a_i - 6400, a_i - 8064)).astype(jnp.float32)
    wdt = jnp.where(lvl0, 80.0, jnp.where(in01, 40.0, 20.0))
    gy = jnp.floor((local + 0.5) / wdt)
    gx = local - gy * wdt
    xc = (gx + 0.5) * stride
    yc = (gy + 0.5) * stride

    xr = fs[0:1, :]
    yr = fs[1:2, :]
    wr = fs[2:3, :]
    hr = fs[3:4, :]
    ob = fs[4:5, :]
    cls = fs[5:85, :]

    bx = (xr + gx) * stride
    by = (yr + gy) * stride
    bw = jnp.exp(wr) * stride
    bh = jnp.exp(hr) * stride

    e = jnp.exp(-jnp.abs(cls))
    l1pe = jnp.log1p(e)
    sbce = jnp.sum(jnp.maximum(cls, 0.0) + l1pe, axis=0, keepdims=True)
    lsc = jnp.minimum(cls, 0.0) - l1pe
    lso = jnp.minimum(ob, 0.0) - jnp.log1p(jnp.exp(-jnp.abs(ob)))
    logs = 0.5 * (lsc + lso)
    s = jnp.exp(logs)
    log1ms = jnp.log(1.0 - s + 1e-8)
    l0 = jnp.sum(log1ms, axis=0, keepdims=True)
    dmat = logs - log1ms

    labv = lab[0]
    gtx = labv[:, 0:1]
    gty = labv[:, 1:2]
    gtw = labv[:, 2:3]
    gth = labv[:, 3:4]

    gcls = labv[:, 4:5]
    c_i = jax.lax.broadcasted_iota(jnp.int32, (_G, _NC), 1)
    onehot = (gcls.astype(jnp.int32) == c_i).astype(jnp.float32)
    dsel = jax.lax.dot_general(onehot, dmat, (((1,), (0,)), ((), ())),
                               precision=jax.lax.Precision.HIGHEST,
                               preferred_element_type=jnp.float32)
    cpsel = jax.lax.dot_general(onehot, cls, (((1,), (0,)), ((), ())),
                                preferred_element_type=jnp.float32)

    in_boxes = ((xc > gtx - 0.5 * gtw) & (xc < gtx + 0.5 * gtw)
                & (yc > gty - 0.5 * gth) & (yc < gty + 0.5 * gth))
    st25 = 2.5 * stride
    in_centers = ((xc + st25 > gtx) & (xc - st25 < gtx)
                  & (yc + st25 > gty) & (yc - st25 < gty))
    fg = (jnp.max((in_boxes | in_centers).astype(jnp.float32), axis=0,
                  keepdims=True) > 0.0) & valid
    in_both = in_boxes & in_centers

    tlx = jnp.maximum(gtx - 0.5 * gtw, bx - 0.5 * bw)
    tly = jnp.maximum(gty - 0.5 * gth, by - 0.5 * bh)
    brx = jnp.minimum(gtx + 0.5 * gtw, bx + 0.5 * bw)
    bry = jnp.minimum(gty + 0.5 * gth, by + 0.5 * bh)
    en = ((tlx < brx) & (tly < bry)).astype(jnp.float32)
    area_i = (brx - tlx) * (bry - tly) * en
    ious = area_i / (gtw * gth + bw * bh - area_i + 1e-16)

    iou_cost = -jnp.log(ious + 1e-8)
    cls_cost = -(l0 + dsel)
    penalty = jnp.where(valid & in_both, 0.0, 100000.0) + jnp.where(
        valid & fg, 0.0, jnp.where(valid, 100000.0, 1e30))
    cost = cls_cost + 3.0 * iou_cost + penalty

    a_f = jax.lax.broadcasted_iota(jnp.int32, (1, _A), 1).astype(jnp.float32)
    cur = jnp.concatenate([jnp.where(fg, ious, 0.0), -cost], axis=0)
    ksum = jnp.zeros((_G, 1), jnp.float32)
    mcs, idcs = [], []
    for _ in range(_NK):
        m = jnp.max(cur, axis=1, keepdims=True)
        ksum = ksum + m[0:_G]
        idx = jnp.min(jnp.where(cur == m, a_f, 3e8), axis=1, keepdims=True)
        mcs.append(m[_G:])
        idcs.append(idx[_G:])
        cur = jnp.where(a_f == idx, -1e35, cur)
    dyn_k = jnp.clip(ksum.astype(jnp.int32), 1, _NK)

    thr_v = jnp.zeros((_G, 1), jnp.float32)
    thr_i = jnp.zeros((_G, 1), jnp.float32)
    for j in range(_NK):
        sel = (dyn_k == j + 1).astype(jnp.float32)
        thr_v = thr_v + sel * (-mcs[j])
        thr_i = thr_i + sel * idcs[j]
    matchf = jnp.where(
        (cost < thr_v) | ((cost == thr_v) & (a_f <= thr_i)), 1.0, 0.0)

    amg = jnp.sum(matchf, axis=0, keepdims=True)
    minc = jnp.min(cost, axis=0, keepdims=True)
    g_f = jax.lax.broadcasted_iota(jnp.int32, (_G, _A), 0).astype(jnp.float32)
    garg = jnp.min(jnp.where(cost == minc, g_f, 99.0), axis=0, keepdims=True)
    matchf = jnp.where(amg > 1.0, jnp.where(g_f == garg, 1.0, 0.0), matchf)

    fgf = jnp.where(amg > 0.0, 1.0, 0.0)
    pious = jnp.sum(matchf * ious, axis=0, keepdims=True)
    csel = jnp.sum(matchf * cpsel, axis=0, keepdims=True)
    tb = jax.lax.dot_general(labv[:, 0:4].T, matchf, (((1,), (0,)), ((), ())),
                             precision=jax.lax.Precision.HIGHEST,
                             preferred_element_type=jnp.float32)
    tx = tb[0:1, :]
    ty = tb[1:2, :]
    tw = tb[2:3, :]
    th = tb[3:4, :]

    tlx2 = jnp.maximum(bx - 0.5 * bw, tx - 0.5 * tw)
    tly2 = jnp.maximum(by - 0.5 * bh, ty - 0.5 * th)
    brx2 = jnp.minimum(bx + 0.5 * bw, tx + 0.5 * tw)
    bry2 = jnp.minimum(by + 0.5 * bh, ty + 0.5 * th)
    en2 = ((tlx2 < brx2) & (tly2 < bry2)).astype(jnp.float32)
    ai2 = (brx2 - tlx2) * (bry2 - tly2) * en2
    iou2 = ai2 / (bw * bh + tw * th - ai2 + 1e-16)
    t_iou = jnp.sum((1.0 - iou2 * iou2) * fgf)

    bce_obj = jnp.maximum(ob, 0.0) - ob * fgf + jnp.log1p(jnp.exp(-jnp.abs(ob)))
    t_obj = jnp.sum(jnp.where(valid, bce_obj, 0.0))
    t_cls = jnp.sum(fgf * (sbce - csel * pious))
    t_fg = jnp.sum(fgf)

    li = jax.lax.broadcasted_iota(jnp.int32, (1, 8), 1)
    vec = (jnp.where(li == 0, t_iou, 0.0) + jnp.where(li == 1, t_obj, 0.0)
           + jnp.where(li == 2, t_cls, 0.0) + jnp.where(li == 3, t_fg, 0.0))

    @pl.when(b == 0)
    def _():
        out[...] = vec

    @pl.when(b > 0)
    def _():
        out[...] = out[...] + vec

    @pl.when(b == _B - 1)
    def _():
        acc = out[...]
        num_fg = jnp.maximum(jnp.sum(jnp.where(li == 3, acc, 0.0)), 1.0)
        total = (5.0 * jnp.sum(jnp.where(li == 0, acc, 0.0))
                 + jnp.sum(jnp.where(li == 1, acc, 0.0))
                 + jnp.sum(jnp.where(li == 2, acc, 0.0)))
        out[...] = jnp.where(li == 4, total / num_fg, acc)


def kernel(feat0, feat1, feat2, labels):
    f0 = feat0.reshape(_B, 85, 6400)
    f1 = feat1.reshape(_B, 85, 1600)
    f2 = feat2.reshape(_B, 85, 400)
    out = pl.pallas_call(
        _loss_body,
        grid=(_B,),
        in_specs=[
            pl.BlockSpec((1, 85, 6400), lambda b: (b, 0, 0)),
            pl.BlockSpec((1, 85, 1600), lambda b: (b, 0, 0)),
            pl.BlockSpec((1, 85, 400), lambda b: (b, 0, 0)),
            pl.BlockSpec((1, _G, 5), lambda b: (b, 0, 0)),
        ],
        out_specs=pl.BlockSpec((1, 8), lambda b: (0, 0)),
        out_shape=jax.ShapeDtypeStruct((1, 8), jnp.float32),
        scratch_shapes=[pltpu.VMEM((85, _A), jnp.float32)],
    )(f0, f1, f2, labels)
    return out[0, 4]


# speedup vs baseline: 1.4151x; 1.0385x over previous
import jax
import jax.numpy as jnp
from jax.experimental import pallas as pl
from jax.experimental.pallas import tpu as pltpu

_NC = 80
_B = 8
_G = 20
_A = 8576
_BIG_I = 1 << 30
_NK = 10


def _loss_body(f0, f1, f2, lab, out, fs):
    b = pl.program_id(0)
    fs[:, 0:6400] = f0[0]
    fs[:, 6400:8000] = f1[0]
    fs[:, 8064:8464] = f2[0]
    fs[:, 8000:8064] = jnp.zeros((85, 64), jnp.float32)
    fs[:, 8464:8576] = jnp.zeros((85, 112), jnp.float32)

    a_i = jax.lax.broadcasted_iota(jnp.int32, (1, _A), 1)
    lvl0 = a_i < 6400
    in01 = a_i < 8064
    valid = lvl0 | ((a_i >= 6400) & (a_i < 8000)) | ((a_i >= 8064) & (a_i < 8464))
    stride = jnp.where(lvl0, 8.0, jnp.where(in01, 16.0, 32.0))
    local = jnp.where(lvl0, a_i, jnp.where(in01, a_i - 6400, a_i - 8064)).astype(jnp.float32)
    wdt = jnp.where(lvl0, 80.0, jnp.where(in01, 40.0, 20.0))
    gy = jnp.floor((local + 0.5) / wdt)
    gx = local - gy * wdt
    xc = (gx + 0.5) * stride
    yc = (gy + 0.5) * stride

    xr = fs[0:1, :]
    yr = fs[1:2, :]
    wr = fs[2:3, :]
    hr = fs[3:4, :]
    ob = fs[4:5, :]
    cls = fs[5:85, :]

    bx = (xr + gx) * stride
    by = (yr + gy) * stride
    bw = jnp.exp(wr) * stride
    bh = jnp.exp(hr) * stride

    e = jnp.exp(-jnp.abs(cls))
    l1pe = jnp.log(1.0 + e)
    sbce = jnp.sum(jnp.maximum(cls, 0.0) + l1pe, axis=0, keepdims=True)
    lsc = jnp.minimum(cls, 0.0) - l1pe
    lso = jnp.minimum(ob, 0.0) - jnp.log(1.0 + jnp.exp(-jnp.abs(ob)))
    logs = 0.5 * (lsc + lso)
    s = jnp.exp(logs)
    log1ms = jnp.log(1.0 - s + 1e-8)
    l0 = jnp.sum(log1ms, axis=0, keepdims=True)
    dmat = logs - log1ms

    labv = lab[0]
    gtx = labv[:, 0:1]
    gty = labv[:, 1:2]
    gtw = labv[:, 2:3]
    gth = labv[:, 3:4]

    gcls = labv[:, 4:5]
    c_i = jax.lax.broadcasted_iota(jnp.int32, (_G, _NC), 1)
    onehot = (gcls.astype(jnp.int32) == c_i).astype(jnp.float32)
    dsel = jax.lax.dot_general(onehot, dmat, (((1,), (0,)), ((), ())),
                               precision=jax.lax.Precision.HIGHEST,
                               preferred_element_type=jnp.float32)
    cpsel = jax.lax.dot_general(onehot, cls, (((1,), (0,)), ((), ())),
                                preferred_element_type=jnp.float32)

    in_boxes = ((xc > gtx - 0.5 * gtw) & (xc < gtx + 0.5 * gtw)
                & (yc > gty - 0.5 * gth) & (yc < gty + 0.5 * gth))
    st25 = 2.5 * stride
    in_centers = ((xc + st25 > gtx) & (xc - st25 < gtx)
                  & (yc + st25 > gty) & (yc - st25 < gty))
    fg = (jnp.max((in_boxes | in_centers).astype(jnp.float32), axis=0,
                  keepdims=True) > 0.0) & valid
    in_both = in_boxes & in_centers

    tlx = jnp.maximum(gtx - 0.5 * gtw, bx - 0.5 * bw)
    tly = jnp.maximum(gty - 0.5 * gth, by - 0.5 * bh)
    brx = jnp.minimum(gtx + 0.5 * gtw, bx + 0.5 * bw)
    bry = jnp.minimum(gty + 0.5 * gth, by + 0.5 * bh)
    en = ((tlx < brx) & (tly < bry)).astype(jnp.float32)
    area_i = (brx - tlx) * (bry - tly) * en
    ious = area_i / (gtw * gth + bw * bh - area_i + 1e-16)

    iou_cost = -jnp.log(ious + 1e-8)
    cls_cost = -(l0 + dsel)
    penalty = jnp.where(valid & in_both, 0.0, 100000.0) + jnp.where(
        valid & fg, 0.0, jnp.where(valid, 100000.0, 1e30))
    cost = cls_cost + 3.0 * iou_cost + penalty

    a_f = jax.lax.broadcasted_iota(jnp.int32, (1, _A), 1).astype(jnp.float32)
    cur = jnp.concatenate([jnp.where(fg, ious, 0.0), -cost], axis=0)
    ksum = jnp.zeros((_G, 1), jnp.float32)
    mcs, idcs = [], []
    for _ in range(_NK):
        m = jnp.max(cur, axis=1, keepdims=True)
        ksum = ksum + m[0:_G]
        idx = jnp.min(jnp.where(cur == m, a_f, 3e8), axis=1, keepdims=True)
        mcs.append(m[_G:])
        idcs.append(idx[_G:])
        cur = jnp.where(a_f == idx, -1e35, cur)
    dyn_k = jnp.clip(ksum.astype(jnp.int32), 1, _NK)

    thr_v = jnp.zeros((_G, 1), jnp.float32)
    thr_i = jnp.zeros((_G, 1), jnp.float32)
    for j in range(_NK):
        sel = (dyn_k == j + 1).astype(jnp.float32)
        thr_v = thr_v + sel * (-mcs[j])
        thr_i = thr_i + sel * idcs[j]
    matchf = jnp.where(
        (cost < thr_v) | ((cost == thr_v) & (a_f <= thr_i)), 1.0, 0.0)

    amg = jnp.sum(matchf, axis=0, keepdims=True)
    minc = jnp.min(cost, axis=0, keepdims=True)
    g_f = jax.lax.broadcasted_iota(jnp.int32, (_G, _A), 0).astype(jnp.float32)
    garg = jnp.min(jnp.where(cost == minc, g_f, 99.0), axis=0, keepdims=True)
    matchf = jnp.where(amg > 1.0, jnp.where(g_f == garg, 1.0, 0.0), matchf)

    fgf = jnp.where(amg > 0.0, 1.0, 0.0)
    pious = jnp.sum(matchf * ious, axis=0, keepdims=True)
    csel = jnp.sum(matchf * cpsel, axis=0, keepdims=True)
    tb = jax.lax.dot_general(labv[:, 0:4].T, matchf, (((1,), (0,)), ((), ())),
                             precision=jax.lax.Precision.HIGHEST,
                             preferred_element_type=jnp.float32)
    tx = tb[0:1, :]
    ty = tb[1:2, :]
    tw = tb[2:3, :]
    th = tb[3:4, :]

    tlx2 = jnp.maximum(bx - 0.5 * bw, tx - 0.5 * tw)
    tly2 = jnp.maximum(by - 0.5 * bh, ty - 0.5 * th)
    brx2 = jnp.minimum(bx + 0.5 * bw, tx + 0.5 * tw)
    bry2 = jnp.minimum(by + 0.5 * bh, ty + 0.5 * th)
    en2 = ((tlx2 < brx2) & (tly2 < bry2)).astype(jnp.float32)
    ai2 = (brx2 - tlx2) * (bry2 - tly2) * en2
    iou2 = ai2 / (bw * bh + tw * th - ai2 + 1e-16)
    t_iou = jnp.sum((1.0 - iou2 * iou2) * fgf)

    bce_obj = jnp.maximum(ob, 0.0) - ob * fgf + jnp.log1p(jnp.exp(-jnp.abs(ob)))
    t_obj = jnp.sum(jnp.where(valid, bce_obj, 0.0))
    t_cls = jnp.sum(fgf * (sbce - csel * pious))
    t_fg = jnp.sum(fgf)

    li = jax.lax.broadcasted_iota(jnp.int32, (1, 8), 1)
    vec = (jnp.where(li == 0, t_iou, 0.0) + jnp.where(li == 1, t_obj, 0.0)
           + jnp.where(li == 2, t_cls, 0.0) + jnp.where(li == 3, t_fg, 0.0))

    @pl.when(b == 0)
    def _():
        out[...] = vec

    @pl.when(b > 0)
    def _():
        out[...] = out[...] + vec

    @pl.when(b == _B - 1)
    def _():
        acc = out[...]
        num_fg = jnp.maximum(jnp.sum(jnp.where(li == 3, acc, 0.0)), 1.0)
        total = (5.0 * jnp.sum(jnp.where(li == 0, acc, 0.0))
                 + jnp.sum(jnp.where(li == 1, acc, 0.0))
                 + jnp.sum(jnp.where(li == 2, acc, 0.0)))
        out[...] = jnp.where(li == 4, total / num_fg, acc)


def kernel(feat0, feat1, feat2, labels):
    f0 = feat0.reshape(_B, 85, 6400)
    f1 = feat1.reshape(_B, 85, 1600)
    f2 = feat2.reshape(_B, 85, 400)
    out = pl.pallas_call(
        _loss_body,
        grid=(_B,),
        in_specs=[
            pl.BlockSpec((1, 85, 6400), lambda b: (b, 0, 0)),
            pl.BlockSpec((1, 85, 1600), lambda b: (b, 0, 0)),
            pl.BlockSpec((1, 85, 400), lambda b: (b, 0, 0)),
            pl.BlockSpec((1, _G, 5), lambda b: (b, 0, 0)),
        ],
        out_specs=pl.BlockSpec((1, 8), lambda b: (0, 0)),
        out_shape=jax.ShapeDtypeStruct((1, 8), jnp.float32),
        scratch_shapes=[pltpu.VMEM((85, _A), jnp.float32)],
    )(f0, f1, f2, labels)
    return out[0, 4]


# speedup vs baseline: 1.4295x; 1.0102x over previous
import jax
import jax.numpy as jnp
from jax.experimental import pallas as pl
from jax.experimental.pallas import tpu as pltpu

_NC = 80
_B = 8
_G = 20
_A = 8576
_NK = 10


def _loss_body(f0, f1, f2, lab, out, fs):
    b = pl.program_id(0)
    fs[:, 0:6400] = f0[0]
    fs[:, 6400:8000] = f1[0]
    fs[:, 8064:8464] = f2[0]
    fs[:, 8000:8064] = jnp.zeros((85, 64), jnp.float32)
    fs[:, 8464:8576] = jnp.zeros((85, 112), jnp.float32)

    a_i = jax.lax.broadcasted_iota(jnp.int32, (1, _A), 1)
    lvl0 = a_i < 6400
    in01 = a_i < 8064
    valid = lvl0 | ((a_i >= 6400) & (a_i < 8000)) | ((a_i >= 8064) & (a_i < 8464))
    stride = jnp.where(lvl0, 8.0, jnp.where(in01, 16.0, 32.0))
    local = jnp.where(lvl0, a_i, jnp.where(in01, a_i - 6400, a_i - 8064)).astype(jnp.float32)
    wdt = jnp.where(lvl0, 80.0, jnp.where(in01, 40.0, 20.0))
    gy = jnp.floor((local + 0.5) / wdt)
    gx = local - gy * wdt
    xc = (gx + 0.5) * stride
    yc = (gy + 0.5) * stride

    xr = fs[0:1, :]
    yr = fs[1:2, :]
    wr = fs[2:3, :]
    hr = fs[3:4, :]
    ob = fs[4:5, :]
    cls = fs[5:85, :]

    bx = (xr + gx) * stride
    by = (yr + gy) * stride
    bw = jnp.exp(wr) * stride
    bh = jnp.exp(hr) * stride

    ones_g = jnp.ones((1, _G), jnp.float32)
    ones_c = jnp.ones((1, _NC), jnp.float32)
    e = jnp.exp(-jnp.abs(cls))
    l1pe = jnp.log(1.0 + e)
    sbce = jax.lax.dot_general(ones_c, jnp.maximum(cls, 0.0) + l1pe,
                               (((1,), (0,)), ((), ())),
                               preferred_element_type=jnp.float32)
    lsc = jnp.minimum(cls, 0.0) - l1pe
    lso = jnp.minimum(ob, 0.0) - jnp.log(1.0 + jnp.exp(-jnp.abs(ob)))
    logs = 0.5 * (lsc + lso)
    s = jnp.exp(logs)
    log1ms = jnp.log(1.0 - s + 1e-8)
    l0 = jnp.sum(log1ms, axis=0, keepdims=True)
    dmat = logs - log1ms

    labv = lab[0]
    gtx = labv[:, 0:1]
    gty = labv[:, 1:2]
    gtw = labv[:, 2:3]
    gth = labv[:, 3:4]

    gcls = labv[:, 4:5]
    c_i = jax.lax.broadcasted_iota(jnp.int32, (_G, _NC), 1)
    onehot = (gcls.astype(jnp.int32) == c_i).astype(jnp.float32)
    dsel = jax.lax.dot_general(onehot, dmat, (((1,), (0,)), ((), ())),
                               precision=jax.lax.Precision.HIGHEST,
                               preferred_element_type=jnp.float32)
    cpsel = jax.lax.dot_general(onehot, cls, (((1,), (0,)), ((), ())),
                                preferred_element_type=jnp.float32)

    in_boxes = ((xc > gtx - 0.5 * gtw) & (xc < gtx + 0.5 * gtw)
                & (yc > gty - 0.5 * gth) & (yc < gty + 0.5 * gth))
    st25 = 2.5 * stride
    in_centers = ((xc + st25 > gtx) & (xc - st25 < gtx)
                  & (yc + st25 > gty) & (yc - st25 < gty))
    orf = jnp.where(in_boxes | in_centers, 1.0, 0.0)
    fg = (jax.lax.dot_general(ones_g, orf, (((1,), (0,)), ((), ())),
                              preferred_element_type=jnp.float32) > 0.0) & valid
    in_both = in_boxes & in_centers

    tlx = jnp.maximum(gtx - 0.5 * gtw, bx - 0.5 * bw)
    tly = jnp.maximum(gty - 0.5 * gth, by - 0.5 * bh)
    brx = jnp.minimum(gtx + 0.5 * gtw, bx + 0.5 * bw)
    bry = jnp.minimum(gty + 0.5 * gth, by + 0.5 * bh)
    en = ((tlx < brx) & (tly < bry)).astype(jnp.float32)
    area_i = (brx - tlx) * (bry - tly) * en
    ious = area_i / (gtw * gth + bw * bh - area_i + 1e-16)

    iou_cost = -jnp.log(ious + 1e-8)
    cls_cost = -(l0 + dsel)
    penalty = jnp.where(valid & in_both, 0.0, 100000.0) + jnp.where(
        valid & fg, 0.0, jnp.where(valid, 100000.0, 1e30))
    cost = cls_cost + 3.0 * iou_cost + penalty

    a_f = jax.lax.broadcasted_iota(jnp.int32, (1, _A), 1).astype(jnp.float32)
    cur = jnp.concatenate([jnp.where(fg, ious, 0.0), -cost], axis=0)
    ksum = jnp.zeros((_G, 1), jnp.float32)
    mcs, idcs = [], []
    for _ in range(_NK):
        m = jnp.max(cur, axis=1, keepdims=True)
        ksum = ksum + m[0:_G]
        idx = jnp.min(jnp.where(cur == m, a_f, 3e8), axis=1, keepdims=True)
        mcs.append(m[_G:])
        idcs.append(idx[_G:])
        cur = jnp.where(a_f == idx, -1e35, cur)
    dyn_k = jnp.clip(ksum.astype(jnp.int32), 1, _NK)

    thr_v = jnp.zeros((_G, 1), jnp.float32)
    thr_i = jnp.zeros((_G, 1), jnp.float32)
    for j in range(_NK):
        sel = (dyn_k == j + 1).astype(jnp.float32)
        thr_v = thr_v + sel * (-mcs[j])
        thr_i = thr_i + sel * idcs[j]
    matchf = jnp.where(
        (cost < thr_v) | ((cost == thr_v) & (a_f <= thr_i)), 1.0, 0.0)

    amg = jax.lax.dot_general(ones_g, matchf, (((1,), (0,)), ((), ())),
                              preferred_element_type=jnp.float32)
    minc = jnp.min(cost, axis=0, keepdims=True)
    g_f = jax.lax.broadcasted_iota(jnp.int32, (_G, _A), 0).astype(jnp.float32)
    garg = jnp.min(jnp.where(cost == minc, g_f, 99.0), axis=0, keepdims=True)
    matchf = jnp.where(amg > 1.0, jnp.where(g_f == garg, 1.0, 0.0), matchf)

    fgf = jnp.where(amg > 0.0, 1.0, 0.0)
    pious = jnp.sum(matchf * ious, axis=0, keepdims=True)
    csel = jnp.sum(matchf * cpsel, axis=0, keepdims=True)
    tb = jax.lax.dot_general(labv[:, 0:4].T, matchf, (((1,), (0,)), ((), ())),
                             precision=jax.lax.Precision.HIGHEST,
                             preferred_element_type=jnp.float32)
    tx = tb[0:1, :]
    ty = tb[1:2, :]
    tw = tb[2:3, :]
    th = tb[3:4, :]

    tlx2 = jnp.maximum(bx - 0.5 * bw, tx - 0.5 * tw)
    tly2 = jnp.maximum(by - 0.5 * bh, ty - 0.5 * th)
    brx2 = jnp.minimum(bx + 0.5 * bw, tx + 0.5 * tw)
    bry2 = jnp.minimum(by + 0.5 * bh, ty + 0.5 * th)
    en2 = ((tlx2 < brx2) & (tly2 < bry2)).astype(jnp.float32)
    ai2 = (brx2 - tlx2) * (bry2 - tly2) * en2
    iou2 = ai2 / (bw * bh + tw * th - ai2 + 1e-16)
    t_iou = jnp.sum((1.0 - iou2 * iou2) * fgf)

    bce_obj = jnp.maximum(ob, 0.0) - ob * fgf + jnp.log1p(jnp.exp(-jnp.abs(ob)))
    t_obj = jnp.sum(jnp.where(valid, bce_obj, 0.0))
    t_cls = jnp.sum(fgf * (sbce - csel * pious))
    t_fg = jnp.sum(fgf)

    li = jax.lax.broadcasted_iota(jnp.int32, (1, 8), 1)
    vec = (jnp.where(li == 0, t_iou, 0.0) + jnp.where(li == 1, t_obj, 0.0)
           + jnp.where(li == 2, t_cls, 0.0) + jnp.where(li == 3, t_fg, 0.0))

    @pl.when(b == 0)
    def _():
        out[...] = vec

    @pl.when(b > 0)
    def _():
        out[...] = out[...] + vec

    @pl.when(b == _B - 1)
    def _():
        acc = out[...]
        num_fg = jnp.maximum(jnp.sum(jnp.where(li == 3, acc, 0.0)), 1.0)
        total = (5.0 * jnp.sum(jnp.where(li == 0, acc, 0.0))
                 + jnp.sum(jnp.where(li == 1, acc, 0.0))
                 + jnp.sum(jnp.where(li == 2, acc, 0.0)))
        out[...] = jnp.where(li == 4, total / num_fg, acc)


def kernel(feat0, feat1, feat2, labels):
    f0 = feat0.reshape(_B, 85, 6400)
    f1 = feat1.reshape(_B, 85, 1600)
    f2 = feat2.reshape(_B, 85, 400)
    out = pl.pallas_call(
        _loss_body,
        grid=(_B,),
        in_specs=[
            pl.BlockSpec((1, 85, 6400), lambda b: (b, 0, 0)),
            pl.BlockSpec((1, 85, 1600), lambda b: (b, 0, 0)),
            pl.BlockSpec((1, 85, 400), lambda b: (b, 0, 0)),
            pl.BlockSpec((1, _G, 5), lambda b: (b, 0, 0)),
        ],
        out_specs=pl.BlockSpec((1, 8), lambda b: (0, 0)),
        out_shape=jax.ShapeDtypeStruct((1, 8), jnp.float32),
        scratch_shapes=[pltpu.VMEM((85, _A), jnp.float32)],
    )(f0, f1, f2, labels)
    return out[0, 4]


# speedup vs baseline: 1.4412x; 1.0081x over previous
import jax
import jax.numpy as jnp
from jax.experimental import pallas as pl
from jax.experimental.pallas import tpu as pltpu

_NC = 80
_B = 8
_G = 20
_A = 8576
_NK = 10


def _loss_body(f0, f1, f2, lab, out, fs):
    b = pl.program_id(0)
    fs[:, 0:6400] = f0[0]
    fs[:, 6400:8000] = f1[0]
    fs[:, 8064:8464] = f2[0]
    fs[:, 8000:8064] = jnp.zeros((85, 64), jnp.float32)
    fs[:, 8464:8576] = jnp.zeros((85, 112), jnp.float32)

    a_i = jax.lax.broadcasted_iota(jnp.int32, (1, _A), 1)
    lvl0 = a_i < 6400
    in01 = a_i < 8064
    valid = lvl0 | ((a_i >= 6400) & (a_i < 8000)) | ((a_i >= 8064) & (a_i < 8464))
    stride = jnp.where(lvl0, 8.0, jnp.where(in01, 16.0, 32.0))
    local = jnp.where(lvl0, a_i, jnp.where(in01, a_i - 6400, a_i - 8064)).astype(jnp.float32)
    wdt = jnp.where(lvl0, 80.0, jnp.where(in01, 40.0, 20.0))
    gy = jnp.floor((local + 0.5) / wdt)
    gx = local - gy * wdt
    xc = (gx + 0.5) * stride
    yc = (gy + 0.5) * stride

    xr = fs[0:1, :]
    yr = fs[1:2, :]
    wr = fs[2:3, :]
    hr = fs[3:4, :]
    ob = fs[4:5, :]
    cls = fs[5:85, :]

    bx = (xr + gx) * stride
    by = (yr + gy) * stride
    bw = jnp.exp(wr) * stride
    bh = jnp.exp(hr) * stride

    ones_g = jnp.ones((1, _G), jnp.float32)
    ones_c = jnp.ones((1, _NC), jnp.float32)
    e = jnp.exp(-jnp.abs(cls))
    l1pe = jnp.log(1.0 + e)
    sbce = jax.lax.dot_general(ones_c, jnp.maximum(cls, 0.0) + l1pe,
                               (((1,), (0,)), ((), ())),
                               preferred_element_type=jnp.float32)
    lsc = jnp.minimum(cls, 0.0) - l1pe
    lso = jnp.minimum(ob, 0.0) - jnp.log(1.0 + jnp.exp(-jnp.abs(ob)))
    logs = 0.5 * (lsc + lso)
    s = jnp.exp(logs)
    log1ms = jnp.log(1.0 - s + 1e-8)
    l0 = jnp.sum(log1ms, axis=0, keepdims=True)
    dmat = logs - log1ms

    labv = lab[0]
    gtx = labv[:, 0:1]
    gty = labv[:, 1:2]
    gtw = labv[:, 2:3]
    gth = labv[:, 3:4]

    gcls = labv[:, 4:5]
    c_i = jax.lax.broadcasted_iota(jnp.int32, (_G, _NC), 1)
    onehot = (gcls.astype(jnp.int32) == c_i).astype(jnp.float32)
    dsel = jax.lax.dot_general(onehot, dmat, (((1,), (0,)), ((), ())),
                               precision=jax.lax.Precision.HIGHEST,
                               preferred_element_type=jnp.float32)
    cpsel = jax.lax.dot_general(onehot, cls, (((1,), (0,)), ((), ())),
                                preferred_element_type=jnp.float32)

    in_boxes = ((xc > gtx - 0.5 * gtw) & (xc < gtx + 0.5 * gtw)
                & (yc > gty - 0.5 * gth) & (yc < gty + 0.5 * gth))
    st25 = 2.5 * stride
    in_centers = ((xc + st25 > gtx) & (xc - st25 < gtx)
                  & (yc + st25 > gty) & (yc - st25 < gty))
    orf = jnp.where(in_boxes | in_centers, 1.0, 0.0)
    fg = (jax.lax.dot_general(ones_g, orf, (((1,), (0,)), ((), ())),
                              preferred_element_type=jnp.float32) > 0.0) & valid
    in_both = in_boxes & in_centers

    tlx = jnp.maximum(gtx - 0.5 * gtw, bx - 0.5 * bw)
    tly = jnp.maximum(gty - 0.5 * gth, by - 0.5 * bh)
    brx = jnp.minimum(gtx + 0.5 * gtw, bx + 0.5 * bw)
    bry = jnp.minimum(gty + 0.5 * gth, by + 0.5 * bh)
    en = ((tlx < brx) & (tly < bry)).astype(jnp.float32)
    area_i = (brx - tlx) * (bry - tly) * en
    ious = area_i / (gtw * gth + bw * bh - area_i + 1e-16)

    iou_cost = -jnp.log(ious + 1e-8)
    cls_cost = -(l0 + dsel)
    penalty = jnp.where(valid & in_both, 0.0, 100000.0) + jnp.where(
        valid & fg, 0.0, jnp.where(valid, 100000.0, 1e30))
    cost = cls_cost + 3.0 * iou_cost + penalty

    a_f = jax.lax.broadcasted_iota(jnp.int32, (1, _A), 1).astype(jnp.float32)
    cur = jnp.concatenate([jnp.where(fg, ious, 0.0), -cost], axis=0)
    ksum = jnp.zeros((_G, 1), jnp.float32)
    mcs, idcs = [], []
    for _ in range(_NK):
        m = jnp.max(cur, axis=1, keepdims=True)
        ksum = ksum + m[0:_G]
        idx = jnp.min(jnp.where(cur == m, a_f, 3e8), axis=1, keepdims=True)
        mcs.append(m[_G:])
        idcs.append(idx[_G:])
        cur = jnp.where(a_f == idx, -1e35, cur)
    dyn_k = jnp.clip(ksum.astype(jnp.int32), 1, _NK)

    thr_v = jnp.zeros((_G, 1), jnp.float32)
    thr_i = jnp.zeros((_G, 1), jnp.float32)
    for j in range(_NK):
        sel = (dyn_k == j + 1).astype(jnp.float32)
        thr_v = thr_v + sel * (-mcs[j])
        thr_i = thr_i + sel * idcs[j]
    matchf = jnp.where(
        (cost < thr_v) | ((cost == thr_v) & (a_f <= thr_i)), 1.0, 0.0)

    amg = jax.lax.dot_general(ones_g, matchf, (((1,), (0,)), ((), ())),
                              preferred_element_type=jnp.float32)
    minc = jnp.min(cost, axis=0, keepdims=True)
    g_f = jax.lax.broadcasted_iota(jnp.int32, (_G, _A), 0).astype(jnp.float32)
    garg = jnp.min(jnp.where(cost == minc, g_f, 99.0), axis=0, keepdims=True)
    matchf = jnp.where(amg > 1.0, jnp.where(g_f == garg, 1.0, 0.0), matchf)

    fgf = jnp.where(amg > 0.0, 1.0, 0.0)
    pious = jax.lax.dot_general(ones_g, matchf * ious, (((1,), (0,)), ((), ())),
                                preferred_element_type=jnp.float32)
    csel = jax.lax.dot_general(ones_g, matchf * cpsel, (((1,), (0,)), ((), ())),
                               preferred_element_type=jnp.float32)
    tb = jax.lax.dot_general(labv[:, 0:4].T, matchf, (((1,), (0,)), ((), ())),
                             precision=jax.lax.Precision.HIGHEST,
                             preferred_element_type=jnp.float32)
    tx = tb[0:1, :]
    ty = tb[1:2, :]
    tw = tb[2:3, :]
    th = tb[3:4, :]

    tlx2 = jnp.maximum(bx - 0.5 * bw, tx - 0.5 * tw)
    tly2 = jnp.maximum(by - 0.5 * bh, ty - 0.5 * th)
    brx2 = jnp.minimum(bx + 0.5 * bw, tx + 0.5 * tw)
    bry2 = jnp.minimum(by + 0.5 * bh, ty + 0.5 * th)
    en2 = ((tlx2 < brx2) & (tly2 < bry2)).astype(jnp.float32)
    ai2 = (brx2 - tlx2) * (bry2 - tly2) * en2
    iou2 = ai2 / (bw * bh + tw * th - ai2 + 1e-16)
    t_iou = jnp.sum((1.0 - iou2 * iou2) * fgf)

    bce_obj = jnp.maximum(ob, 0.0) - ob * fgf + jnp.log1p(jnp.exp(-jnp.abs(ob)))
    t_obj = jnp.sum(jnp.where(valid, bce_obj, 0.0))
    t_cls = jnp.sum(fgf * (sbce - csel * pious))
    t_fg = jnp.sum(fgf)

    li = jax.lax.broadcasted_iota(jnp.int32, (1, 8), 1)
    vec = (jnp.where(li == 0, t_iou, 0.0) + jnp.where(li == 1, t_obj, 0.0)
           + jnp.where(li == 2, t_cls, 0.0) + jnp.where(li == 3, t_fg, 0.0))

    @pl.when(b == 0)
    def _():
        out[...] = vec

    @pl.when(b > 0)
    def _():
        out[...] = out[...] + vec

    @pl.when(b == _B - 1)
    def _():
        acc = out[...]
        num_fg = jnp.maximum(jnp.sum(jnp.where(li == 3, acc, 0.0)), 1.0)
        total = (5.0 * jnp.sum(jnp.where(li == 0, acc, 0.0))
                 + jnp.sum(jnp.where(li == 1, acc, 0.0))
                 + jnp.sum(jnp.where(li == 2, acc, 0.0)))
        out[...] = jnp.where(li == 4, total / num_fg, acc)


def kernel(feat0, feat1, feat2, labels):
    f0 = feat0.reshape(_B, 85, 6400)
    f1 = feat1.reshape(_B, 85, 1600)
    f2 = feat2.reshape(_B, 85, 400)
    out = pl.pallas_call(
        _loss_body,
        grid=(_B,),
        in_specs=[
            pl.BlockSpec((1, 85, 6400), lambda b: (b, 0, 0)),
            pl.BlockSpec((1, 85, 1600), lambda b: (b, 0, 0)),
            pl.BlockSpec((1, 85, 400), lambda b: (b, 0, 0)),
            pl.BlockSpec((1, _G, 5), lambda b: (b, 0, 0)),
        ],
        out_specs=pl.BlockSpec((1, 8), lambda b: (0, 0)),
        out_shape=jax.ShapeDtypeStruct((1, 8), jnp.float32),
        scratch_shapes=[pltpu.VMEM((85, _A), jnp.float32)],
    )(f0, f1, f2, labels)
    return out[0, 4]
